# baseline (speedup 1.0000x reference)
import functools

import jax
import jax.numpy as jnp
from jax import lax
from jax.experimental import pallas as pl
from jax.experimental.pallas import tpu as pltpu
from jax.experimental.pallas import tpu_sc as plsc

N = 4096
L = 16
NC = 2
NS = 16
NW = NC * NS
RPW = N // NW
NCHUNK = N // L
BIG = 3.0e38


def _prep_body(pose_ref, kb_ref, pr_ref, coords_ref, diag_ref):
    px = pr_ref[0, :]
    py = pr_ref[1, :]
    pz = pr_ref[2, :]
    coords_ref[0, :] = px
    coords_ref[1, :] = py
    coords_ref[2, :] = pz
    for b in range(4):
        s = b * 1024
        kx = kb_ref[0, pl.ds(s, 1024)]
        ky = kb_ref[1, pl.ds(s, 1024)]
        kz = kb_ref[2, pl.ds(s, 1024)]
        for d in range(3):
            g = (pose_ref[b, d * 4 + 0] * kx
                 + pose_ref[b, d * 4 + 1] * ky
                 + pose_ref[b, d * 4 + 2] * kz
                 + pose_ref[b, d * 4 + 3])
            coords_ref[3 + d, pl.ds(s, 1024)] = g
    gx = coords_ref[3, :]
    gy = coords_ref[4, :]
    gz = coords_ref[5, :]
    dx = px - gx
    dy = py - gy
    dz = pz - gz
    diag_ref[0, :] = dx * dx + dy * dy + dz * dz
    diag_ref[1, :] = jnp.abs(dx) + jnp.abs(dy) + jnp.abs(dz)


_prep_call = pl.pallas_call(
    _prep_body,
    out_shape=[
        jax.ShapeDtypeStruct((6, N), jnp.float32),
        jax.ShapeDtypeStruct((2, N), jnp.float32),
    ],
    in_specs=[
        pl.BlockSpec(memory_space=pltpu.SMEM),
        pl.BlockSpec(memory_space=pltpu.VMEM),
        pl.BlockSpec(memory_space=pltpu.VMEM),
    ],
)


def _sc_body(coords, minlt_hbm, mingt_hbm, gx, gy, gz, px, py, pz, lt_o, gt_o):
    wid = lax.axis_index("s") * NC + lax.axis_index("c")
    base = wid * RPW
    pltpu.sync_copy(coords.at[3], gx)
    pltpu.sync_copy(coords.at[4], gy)
    pltpu.sync_copy(coords.at[5], gz)
    pltpu.sync_copy(coords.at[0, pl.ds(base, RPW)], px)
    pltpu.sync_copy(coords.at[1, pl.ds(base, RPW)], py)
    pltpu.sync_copy(coords.at[2, pl.ds(base, RPW)], pz)

    lane = lax.iota(jnp.int32, L)
    big = jnp.full((L,), BIG, jnp.float32)
    zeros = jnp.zeros((L,), jnp.float32)

    def group_fn(g, carry):
        gbase = g * L
        pxc = px[pl.ds(gbase, L)]
        pyc = py[pl.ds(gbase, L)]
        pzc = pz[pl.ds(gbase, L)]
        cb = (base + gbase) // L

        def row_fn(rr, rcarry):
            res_lt, res_gt = rcarry
            sel = lane == rr
            pxs = jnp.full((L,), jnp.sum(jnp.where(sel, pxc, 0.0)))
            pys = jnp.full((L,), jnp.sum(jnp.where(sel, pyc, 0.0)))
            pzs = jnp.full((L,), jnp.sum(jnp.where(sel, pzc, 0.0)))

            def dsq_at(c):
                gxv = gx[pl.ds(c * L, L)]
                gyv = gy[pl.ds(c * L, L)]
                gzv = gz[pl.ds(c * L, L)]
                dx = pxs - gxv
                dy = pys - gyv
                dz = pzs - gzv
                return dx * dx + dy * dy + dz * dz

            def mn(c, acc):
                return jnp.minimum(acc, dsq_at(c))

            acc_lt = lax.fori_loop(0, cb, mn, big)
            acc_gt = lax.fori_loop(cb + 1, NCHUNK, mn, big)
            db = dsq_at(cb)
            acc_lt = jnp.minimum(acc_lt, jnp.where(lane < rr, db, BIG))
            acc_gt = jnp.minimum(acc_gt, jnp.where(lane > rr, db, BIG))
            mlt = jnp.min(acc_lt)
            mgt = jnp.min(acc_gt)
            res_lt = jnp.where(sel, mlt, res_lt)
            res_gt = jnp.where(sel, mgt, res_gt)
            return res_lt, res_gt

        res_lt, res_gt = lax.fori_loop(0, L, row_fn, (zeros, zeros))
        lt_o[pl.ds(gbase, L)] = res_lt
        gt_o[pl.ds(gbase, L)] = res_gt
        return carry

    lax.fori_loop(0, RPW // L, group_fn, 0)
    pltpu.sync_copy(lt_o, minlt_hbm.at[pl.ds(base, RPW)])
    pltpu.sync_copy(gt_o, mingt_hbm.at[pl.ds(base, RPW)])


@functools.cache
def _get_sc_call():
    mesh = plsc.VectorSubcoreMesh(core_axis_name="c", subcore_axis_name="s")
    return functools.partial(
        pl.kernel,
        out_type=[
            jax.ShapeDtypeStruct((N,), jnp.float32),
            jax.ShapeDtypeStruct((N,), jnp.float32),
        ],
        mesh=mesh,
        compiler_params=pltpu.CompilerParams(needs_layout_passes=False),
        scratch_types=[
            pltpu.VMEM((N,), jnp.float32),
            pltpu.VMEM((N,), jnp.float32),
            pltpu.VMEM((N,), jnp.float32),
            pltpu.VMEM((RPW,), jnp.float32),
            pltpu.VMEM((RPW,), jnp.float32),
            pltpu.VMEM((RPW,), jnp.float32),
            pltpu.VMEM((RPW,), jnp.float32),
            pltpu.VMEM((RPW,), jnp.float32),
        ],
    )(_sc_body)


def _post_body(mlt_ref, mgt_ref, diag_ref, w_ref, lg_ref, out_ref):
    dii = diag_ref[0]
    err = diag_ref[1]
    s_lt = jnp.sqrt(mlt_ref[...])
    s_gt = jnp.sqrt(mgt_ref[...])
    s_ii = jnp.sqrt(dii)
    m = jnp.logical_and(s_lt > s_ii, s_gt >= s_ii).astype(jnp.float32)
    x = lg_ref[...]
    bce0 = jnp.maximum(x, 0.0) + jnp.log(1.0 + jnp.exp(-jnp.abs(x)))
    bce1 = bce0 - x
    cnt1 = jnp.sum(m)
    cnt0 = jnp.float32(N) - cnt1
    mean0 = jnp.sum(bce0 * (1.0 - m)) / jnp.maximum(cnt0, 1.0)
    mean1 = jnp.sum(bce1 * m) / jnp.maximum(cnt1, 1.0)
    inlier = (jnp.where(cnt0 > 0.0, mean0, 0.0)
              + jnp.where(cnt1 > 0.0, mean1, 0.0)) * 0.5
    w = w_ref[...]
    mean_err = jnp.sum(w * err) / jnp.maximum(jnp.sum(w), 1e-6)
    out_ref[0, 0] = mean_err + inlier


_post_call = pl.pallas_call(
    _post_body,
    out_shape=jax.ShapeDtypeStruct((1, 1), jnp.float32),
    out_specs=pl.BlockSpec(memory_space=pltpu.SMEM),
)


def kernel(kp_before, kp_warped_pred, pose_gt, overlap_weights, inlier_logits):
    kb = jnp.transpose(kp_before, (2, 0, 1)).reshape(3, N)
    pr = jnp.transpose(kp_warped_pred, (2, 0, 1)).reshape(3, N)
    pose = pose_gt.reshape(4, 12)
    coords, diag = _prep_call(pose, kb, pr)
    minlt, mingt = _get_sc_call()(coords)
    out = _post_call(
        minlt.reshape(32, 128),
        mingt.reshape(32, 128),
        diag.reshape(2, 32, 128),
        overlap_weights.reshape(32, 128),
        inlier_logits.reshape(32, 128),
    )
    return out[0, 0]


# speedup vs baseline: 1.6714x; 1.6714x over previous
import functools

import jax
import jax.numpy as jnp
from jax import lax
from jax.experimental import pallas as pl
from jax.experimental.pallas import tpu as pltpu
from jax.experimental.pallas import tpu_sc as plsc

N = 4096
L = 16
NC = 2
NS = 16
NW = NC * NS
RPW = N // NW
NCHUNK = N // L
BIG = 3.0e38


def _prep_body(pose_ref, kb_ref, pr_ref, coords_ref, diag_ref):
    px = pr_ref[0, :]
    py = pr_ref[1, :]
    pz = pr_ref[2, :]
    coords_ref[0, :] = px
    coords_ref[1, :] = py
    coords_ref[2, :] = pz
    for b in range(4):
        s = b * 1024
        kx = kb_ref[0, pl.ds(s, 1024)]
        ky = kb_ref[1, pl.ds(s, 1024)]
        kz = kb_ref[2, pl.ds(s, 1024)]
        for d in range(3):
            g = (pose_ref[b, d * 4 + 0] * kx
                 + pose_ref[b, d * 4 + 1] * ky
                 + pose_ref[b, d * 4 + 2] * kz
                 + pose_ref[b, d * 4 + 3])
            coords_ref[3 + d, pl.ds(s, 1024)] = g
    gx = coords_ref[3, :]
    gy = coords_ref[4, :]
    gz = coords_ref[5, :]
    dx = px - gx
    dy = py - gy
    dz = pz - gz
    diag_ref[0, :] = dx * dx + dy * dy + dz * dz
    diag_ref[1, :] = jnp.abs(dx) + jnp.abs(dy) + jnp.abs(dz)


_prep_call = pl.pallas_call(
    _prep_body,
    out_shape=[
        jax.ShapeDtypeStruct((6, N), jnp.float32),
        jax.ShapeDtypeStruct((2, N), jnp.float32),
    ],
    in_specs=[
        pl.BlockSpec(memory_space=pltpu.SMEM),
        pl.BlockSpec(memory_space=pltpu.VMEM),
        pl.BlockSpec(memory_space=pltpu.VMEM),
    ],
)


def _sc_body(coords, minlt_hbm, mingt_hbm, gx, gy, gz, px, py, pz, lt_o, gt_o):
    wid = lax.axis_index("s") * NC + lax.axis_index("c")
    base = wid * RPW
    pltpu.sync_copy(coords.at[3], gx)
    pltpu.sync_copy(coords.at[4], gy)
    pltpu.sync_copy(coords.at[5], gz)
    pltpu.sync_copy(coords.at[0, pl.ds(base, RPW)], px)
    pltpu.sync_copy(coords.at[1, pl.ds(base, RPW)], py)
    pltpu.sync_copy(coords.at[2, pl.ds(base, RPW)], pz)

    lane = lax.iota(jnp.int32, L)
    big = jnp.full((L,), BIG, jnp.float32)
    zeros = jnp.zeros((L,), jnp.float32)

    def group_fn(g, carry):
        gbase = g * L
        pxc = px[pl.ds(gbase, L)]
        pyc = py[pl.ds(gbase, L)]
        pzc = pz[pl.ds(gbase, L)]
        cb = (base + gbase) // L

        def row_fn(rr, rcarry):
            res_lt, res_gt = rcarry
            sel = lane == rr
            pxs = jnp.full((L,), jnp.sum(jnp.where(sel, pxc, 0.0)))
            pys = jnp.full((L,), jnp.sum(jnp.where(sel, pyc, 0.0)))
            pzs = jnp.full((L,), jnp.sum(jnp.where(sel, pzc, 0.0)))

            def dsq_at(c):
                gxv = gx[pl.ds(c * L, L)]
                gyv = gy[pl.ds(c * L, L)]
                gzv = gz[pl.ds(c * L, L)]
                dx = pxs - gxv
                dy = pys - gyv
                dz = pzs - gzv
                return dx * dx + dy * dy + dz * dz

            def mn(c, acc):
                return jnp.minimum(acc, dsq_at(c))

            acc_lt = plsc.parallel_loop(0, cb, unroll=8, carry=big)(mn)
            acc_gt = plsc.parallel_loop(cb + 1, NCHUNK, unroll=8, carry=big)(mn)
            db = dsq_at(cb)
            acc_lt = jnp.minimum(acc_lt, jnp.where(lane < rr, db, BIG))
            acc_gt = jnp.minimum(acc_gt, jnp.where(lane > rr, db, BIG))
            mlt = jnp.min(acc_lt)
            mgt = jnp.min(acc_gt)
            res_lt = jnp.where(sel, mlt, res_lt)
            res_gt = jnp.where(sel, mgt, res_gt)
            return res_lt, res_gt

        res_lt, res_gt = lax.fori_loop(0, L, row_fn, (zeros, zeros))
        lt_o[pl.ds(gbase, L)] = res_lt
        gt_o[pl.ds(gbase, L)] = res_gt
        return carry

    lax.fori_loop(0, RPW // L, group_fn, 0)
    pltpu.sync_copy(lt_o, minlt_hbm.at[pl.ds(base, RPW)])
    pltpu.sync_copy(gt_o, mingt_hbm.at[pl.ds(base, RPW)])


@functools.cache
def _get_sc_call():
    mesh = plsc.VectorSubcoreMesh(core_axis_name="c", subcore_axis_name="s")
    return functools.partial(
        pl.kernel,
        out_type=[
            jax.ShapeDtypeStruct((N,), jnp.float32),
            jax.ShapeDtypeStruct((N,), jnp.float32),
        ],
        mesh=mesh,
        compiler_params=pltpu.CompilerParams(needs_layout_passes=False),
        scratch_types=[
            pltpu.VMEM((N,), jnp.float32),
            pltpu.VMEM((N,), jnp.float32),
            pltpu.VMEM((N,), jnp.float32),
            pltpu.VMEM((RPW,), jnp.float32),
            pltpu.VMEM((RPW,), jnp.float32),
            pltpu.VMEM((RPW,), jnp.float32),
            pltpu.VMEM((RPW,), jnp.float32),
            pltpu.VMEM((RPW,), jnp.float32),
        ],
    )(_sc_body)


def _post_body(mlt_ref, mgt_ref, diag_ref, w_ref, lg_ref, out_ref):
    dii = diag_ref[0]
    err = diag_ref[1]
    s_lt = jnp.sqrt(mlt_ref[...])
    s_gt = jnp.sqrt(mgt_ref[...])
    s_ii = jnp.sqrt(dii)
    m = jnp.logical_and(s_lt > s_ii, s_gt >= s_ii).astype(jnp.float32)
    x = lg_ref[...]
    bce0 = jnp.maximum(x, 0.0) + jnp.log(1.0 + jnp.exp(-jnp.abs(x)))
    bce1 = bce0 - x
    cnt1 = jnp.sum(m)
    cnt0 = jnp.float32(N) - cnt1
    mean0 = jnp.sum(bce0 * (1.0 - m)) / jnp.maximum(cnt0, 1.0)
    mean1 = jnp.sum(bce1 * m) / jnp.maximum(cnt1, 1.0)
    inlier = (jnp.where(cnt0 > 0.0, mean0, 0.0)
              + jnp.where(cnt1 > 0.0, mean1, 0.0)) * 0.5
    w = w_ref[...]
    mean_err = jnp.sum(w * err) / jnp.maximum(jnp.sum(w), 1e-6)
    out_ref[0, 0] = mean_err + inlier


_post_call = pl.pallas_call(
    _post_body,
    out_shape=jax.ShapeDtypeStruct((1, 1), jnp.float32),
    out_specs=pl.BlockSpec(memory_space=pltpu.SMEM),
)


def kernel(kp_before, kp_warped_pred, pose_gt, overlap_weights, inlier_logits):
    kb = jnp.transpose(kp_before, (2, 0, 1)).reshape(3, N)
    pr = jnp.transpose(kp_warped_pred, (2, 0, 1)).reshape(3, N)
    pose = pose_gt.reshape(4, 12)
    coords, diag = _prep_call(pose, kb, pr)
    minlt, mingt = _get_sc_call()(coords)
    out = _post_call(
        minlt.reshape(32, 128),
        mingt.reshape(32, 128),
        diag.reshape(2, 32, 128),
        overlap_weights.reshape(32, 128),
        inlier_logits.reshape(32, 128),
    )
    return out[0, 0]
